# baseline (speedup 1.0000x reference)
import functools
import numpy as np
import jax
import jax.numpy as jnp
from jax import lax
from jax.experimental import pallas as pl
from jax.experimental.pallas import tpu as pltpu
from jax.experimental.pallas import tpu_sc as plsc

VOCAB = 100000
D_MODEL = 64
BATCH = 1024
SEQ_LEN = 200

_NC = 2
_NS = 16
_NW = _NC * _NS
_BB = BATCH // 128
_UNITS = SEQ_LEN * _BB
_UPW = _UNITS // _NW
_L = 16
_XROWS = 256


def _positional_encoding(length, depth):
    half = depth / 2
    positions = np.arange(length)[:, np.newaxis]
    depths = np.arange(half)[np.newaxis, :] / half
    angle_rates = 1 / 10000 ** depths
    angle_rads = positions * angle_rates
    pos = np.concatenate([np.sin(angle_rads), np.cos(angle_rads)], axis=-1)
    return pos.astype(np.float32)


def _sc_body(table_hbm, xT_hbm, posP_hbm, out_hbm,
             xall_v, rows_v, outb_v, pos_v, gsems, ssems):
    wid = lax.axis_index("s") * _NC + lax.axis_index("c")
    u0 = wid * _UPW
    l0 = u0 // _BB

    pltpu.sync_copy(xT_hbm.at[pl.ds(l0, 8)], xall_v)
    pltpu.sync_copy(posP_hbm.at[pl.ds(l0, 8)], pos_v)

    def unit_pos(u):
        return u // _BB, lax.rem(u, _BB)

    def gather(u, b):
        l, bt = unit_pos(u)
        return pltpu.make_async_copy(
            table_hbm.at[xall_v.at[l - l0, pl.ds(bt * 128, 128)]],
            rows_v.at[b],
            gsems[b],
        )

    def store(u, b, dt):
        l, bt = unit_pos(u)
        return pltpu.make_async_copy(
            outb_v.at[b, pl.ds(dt * 8, 8)],
            out_hbm.at[l, dt, bt],
            ssems[b],
        )

    def compute(u, b):
        l, bt = unit_pos(u)
        lrow = l - l0
        rows = rows_v.at[b]

        def chunk_body(ch, carry):
            pv = pos_v[lrow, pl.ds(ch * _L, _L)]
            for g in range(8):
                sl = pl.ds(g * _L, _L)
                row_g = g * _L + lax.iota(jnp.int32, _L)
                for dd in range(_L):
                    d = ch * _L + dd
                    dvec = jnp.full((_L,), 0, jnp.int32) + d
                    vals = plsc.load_gather(rows, [row_g, dvec])
                    outb_v[b, d, sl] = vals * 8.0 + pv[dd]
            return carry

        lax.fori_loop(0, D_MODEL // _L, chunk_body, 0)

    gather(u0, 0).start()
    gather(u0 + 1, 1).start()

    def outer(i2, carry):
        for b in range(2):
            ul = i2 * 2 + b
            u = u0 + ul
            gather(u, b).wait()

            @pl.when(ul >= 2)
            def _():
                for dt in range(8):
                    store(u0, b, 0).wait()

            compute(u, b)
            for dt in range(8):
                store(u, b, dt).start()

            @pl.when(ul + 2 < _UPW)
            def _():
                gather(u + 2, b).start()

        return carry

    lax.fori_loop(0, _UPW // 2, outer, 0)
    for b in range(2):
        for dt in range(8):
            store(u0, b, 0).wait()


@jax.jit
def _pos_embed(table, xT, posP):
    mesh = plsc.VectorSubcoreMesh(
        core_axis_name="c", subcore_axis_name="s", num_cores=_NC
    )
    k = pl.kernel(
        _sc_body,
        out_type=jax.ShapeDtypeStruct((SEQ_LEN, 8, 8, 8, 128), jnp.float32),
        mesh=mesh,
        scratch_types=[
            pltpu.VMEM((8, 1024), jnp.int32),
            pltpu.VMEM((2, 128, D_MODEL), jnp.float32),
            pltpu.VMEM((2, D_MODEL, 128), jnp.float32),
            pltpu.VMEM((8, 128), jnp.float32),
            [pltpu.SemaphoreType.DMA] * 2,
            [pltpu.SemaphoreType.DMA] * 2,
        ],
        compiler_params=pltpu.CompilerParams(
            use_tc_tiling_on_sc=False, needs_layout_passes=False
        ),
    )
    return k(table, xT, posP)


def kernel(x, table):
    pos = _positional_encoding(SEQ_LEN, D_MODEL)
    posP = np.zeros((_XROWS, 128), np.float32)
    posP[:SEQ_LEN, :D_MODEL] = pos
    posP = jnp.asarray(posP)
    xT = jnp.pad(jnp.transpose(x.astype(jnp.int32)),
                 ((0, _XROWS - SEQ_LEN), (0, 0)))
    out5 = _pos_embed(table, xT, posP)
    return jnp.transpose(out5, (2, 4, 0, 1, 3)).reshape(BATCH, SEQ_LEN, D_MODEL)


# speedup vs baseline: 1.1546x; 1.1546x over previous
import functools
import numpy as np
import jax
import jax.numpy as jnp
from jax import lax
from jax.experimental import pallas as pl
from jax.experimental.pallas import tpu as pltpu
from jax.experimental.pallas import tpu_sc as plsc

VOCAB = 100000
D_MODEL = 64
BATCH = 1024
SEQ_LEN = 200

_NC = 2
_NS = 16
_NW = _NC * _NS
_BB = BATCH // 128
_UNITS = SEQ_LEN * _BB
_UPW = _UNITS // _NW
_L = 16
_XROWS = 256


def _positional_encoding(length, depth):
    half = depth / 2
    positions = np.arange(length)[:, np.newaxis]
    depths = np.arange(half)[np.newaxis, :] / half
    angle_rates = 1 / 10000 ** depths
    angle_rads = positions * angle_rates
    pos = np.concatenate([np.sin(angle_rads), np.cos(angle_rads)], axis=-1)
    return pos.astype(np.float32)


def _sc_body(table_hbm, xT_hbm, posP_hbm, out_hbm,
             xall_v, rows_v, outb_v, pos_v, gsems, ssems):
    wid = lax.axis_index("s") * _NC + lax.axis_index("c")
    u0 = wid * _UPW
    l0 = u0 // _BB

    pltpu.sync_copy(xT_hbm.at[pl.ds(l0, 8)], xall_v)
    pltpu.sync_copy(posP_hbm.at[pl.ds(l0, 8)], pos_v)

    def unit_pos(u):
        return u // _BB, lax.rem(u, _BB)

    def gather(u, b):
        l, bt = unit_pos(u)
        return pltpu.make_async_copy(
            table_hbm.at[xall_v.at[l - l0, pl.ds(bt * 128, 128)]],
            rows_v.at[b],
            gsems[b],
        )

    def store(u, b, dt):
        l, bt = unit_pos(u)
        return pltpu.make_async_copy(
            outb_v.at[b, pl.ds(dt * 8, 8)],
            out_hbm.at[l, dt, bt],
            ssems[b],
        )

    def compute(u, b):
        l, bt = unit_pos(u)
        lrow = l - l0
        outb = outb_v.at[b]

        iota = lax.iota(jnp.int32, _L)
        carry0 = tuple(
            pos_v[lrow, pl.ds(k * _L, _L)] for k in range(4)
        ) + tuple(k * _L + iota for k in range(4))

        def r_body(r, c):
            rvec = jnp.zeros((_L,), jnp.int32) + r
            for k in range(4):
                vals = rows_v[b, r, pl.ds(k * _L, _L)] * 8.0 + c[k]
                plsc.store_scatter(outb, [c[4 + k], rvec], vals)
            return c

        lax.fori_loop(0, 128, r_body, carry0, unroll=4)

    gather(u0, 0).start()
    gather(u0 + 1, 1).start()

    def outer(i2, carry):
        for b in range(2):
            ul = i2 * 2 + b
            u = u0 + ul
            gather(u, b).wait()

            @pl.when(ul >= 2)
            def _():
                for dt in range(8):
                    store(u0, b, 0).wait()

            compute(u, b)
            for dt in range(8):
                store(u, b, dt).start()

            @pl.when(ul + 2 < _UPW)
            def _():
                gather(u + 2, b).start()

        return carry

    lax.fori_loop(0, _UPW // 2, outer, 0)
    for b in range(2):
        for dt in range(8):
            store(u0, b, 0).wait()


@jax.jit
def _pos_embed(table, xT, posP):
    mesh = plsc.VectorSubcoreMesh(
        core_axis_name="c", subcore_axis_name="s", num_cores=_NC
    )
    k = pl.kernel(
        _sc_body,
        out_type=jax.ShapeDtypeStruct((SEQ_LEN, 8, 8, 8, 128), jnp.float32),
        mesh=mesh,
        scratch_types=[
            pltpu.VMEM((8, 1024), jnp.int32),
            pltpu.VMEM((2, 128, D_MODEL), jnp.float32),
            pltpu.VMEM((2, D_MODEL, 128), jnp.float32),
            pltpu.VMEM((8, 128), jnp.float32),
            [pltpu.SemaphoreType.DMA] * 2,
            [pltpu.SemaphoreType.DMA] * 2,
        ],
        compiler_params=pltpu.CompilerParams(
            use_tc_tiling_on_sc=False, needs_layout_passes=False
        ),
    )
    return k(table, xT, posP)


def kernel(x, table):
    pos = _positional_encoding(SEQ_LEN, D_MODEL)
    posP = np.zeros((_XROWS, 128), np.float32)
    posP[:SEQ_LEN, :D_MODEL] = pos
    posP = jnp.asarray(posP)
    xT = jnp.pad(jnp.transpose(x.astype(jnp.int32)),
                 ((0, _XROWS - SEQ_LEN), (0, 0)))
    out5 = _pos_embed(table, xT, posP)
    return jnp.transpose(out5, (2, 4, 0, 1, 3)).reshape(BATCH, SEQ_LEN, D_MODEL)


# speedup vs baseline: 1.7275x; 1.4962x over previous
import functools
import numpy as np
import jax
import jax.numpy as jnp
from jax import lax
from jax.experimental import pallas as pl
from jax.experimental.pallas import tpu as pltpu
from jax.experimental.pallas import tpu_sc as plsc

VOCAB = 100000
D_MODEL = 64
BATCH = 1024
SEQ_LEN = 200

_NC = 2
_NS = 16
_NW = _NC * _NS
_BB = BATCH // 128
_UNITS = SEQ_LEN * _BB
_UPW = _UNITS // _NW
_L = 16
_XROWS = 256


def _positional_encoding(length, depth):
    half = depth / 2
    positions = np.arange(length)[:, np.newaxis]
    depths = np.arange(half)[np.newaxis, :] / half
    angle_rates = 1 / 10000 ** depths
    angle_rads = positions * angle_rates
    pos = np.concatenate([np.sin(angle_rads), np.cos(angle_rads)], axis=-1)
    return pos.astype(np.float32)


def _sc_body(table_hbm, xT_hbm, posP_hbm, out_hbm,
             xall_v, rows_v, outb_v, pos_v, gsems, ssems):
    wid = lax.axis_index("s") * _NC + lax.axis_index("c")
    u0 = wid * _UPW
    l0 = u0 // _BB

    pltpu.sync_copy(xT_hbm.at[pl.ds(l0, 8)], xall_v)
    pltpu.sync_copy(posP_hbm.at[pl.ds(l0, 8)], pos_v)

    def unit_pos(u):
        return u // _BB, lax.rem(u, _BB)

    def gather(u, b):
        l, bt = unit_pos(u)
        return pltpu.make_async_copy(
            table_hbm.at[xall_v.at[l - l0, pl.ds(bt * 128, 128)]],
            rows_v.at[b],
            gsems[b],
        )

    def store(u, b, dt):
        l, bt = unit_pos(u)
        return pltpu.make_async_copy(
            outb_v.at[b, pl.ds(dt * 8, 8)],
            out_hbm.at[l, dt, bt],
            ssems[b],
        )

    def compute(u, b):
        l, bt = unit_pos(u)
        lrow = l - l0
        outb = outb_v.at[b]
        rows = rows_v.at[b]

        iota = lax.iota(jnp.int32, _L)
        rots = [
            lax.bitwise_and(iota + s, _L - 1) for s in range(_L)
        ]

        def db_body(db, carry):
            d0 = db * _L
            colvec = d0 + iota
            posvec = pos_v[lrow, pl.ds(d0, _L)]
            for rb in range(8):
                r0 = rb * _L
                for s in range(_L):
                    rvec = r0 + rots[s]
                    vals = plsc.load_gather(rows, [rvec, colvec])
                    plsc.store_scatter(
                        outb, [colvec, rvec], vals * 8.0 + posvec
                    )
            return carry

        lax.fori_loop(0, D_MODEL // _L, db_body, 0)

    gather(u0, 0).start()
    gather(u0 + 1, 1).start()

    def outer(i2, carry):
        for b in range(2):
            ul = i2 * 2 + b
            u = u0 + ul
            gather(u, b).wait()

            @pl.when(ul >= 2)
            def _():
                for dt in range(8):
                    store(u0, b, 0).wait()

            compute(u, b)
            for dt in range(8):
                store(u, b, dt).start()

            @pl.when(ul + 2 < _UPW)
            def _():
                gather(u + 2, b).start()

        return carry

    lax.fori_loop(0, _UPW // 2, outer, 0)
    for b in range(2):
        for dt in range(8):
            store(u0, b, 0).wait()


@jax.jit
def _pos_embed(table, xT, posP):
    mesh = plsc.VectorSubcoreMesh(
        core_axis_name="c", subcore_axis_name="s", num_cores=_NC
    )
    k = pl.kernel(
        _sc_body,
        out_type=jax.ShapeDtypeStruct((SEQ_LEN, 8, 8, 8, 128), jnp.float32),
        mesh=mesh,
        scratch_types=[
            pltpu.VMEM((8, 1024), jnp.int32),
            pltpu.VMEM((2, 128, D_MODEL), jnp.float32),
            pltpu.VMEM((2, D_MODEL, 128), jnp.float32),
            pltpu.VMEM((8, 128), jnp.float32),
            [pltpu.SemaphoreType.DMA] * 2,
            [pltpu.SemaphoreType.DMA] * 2,
        ],
        compiler_params=pltpu.CompilerParams(
            use_tc_tiling_on_sc=False, needs_layout_passes=False
        ),
    )
    return k(table, xT, posP)


def kernel(x, table):
    pos = _positional_encoding(SEQ_LEN, D_MODEL)
    posP = np.zeros((_XROWS, 128), np.float32)
    posP[:SEQ_LEN, :D_MODEL] = pos
    posP = jnp.asarray(posP)
    xT = jnp.pad(jnp.transpose(x.astype(jnp.int32)),
                 ((0, _XROWS - SEQ_LEN), (0, 0)))
    out5 = _pos_embed(table, xT, posP)
    return jnp.transpose(out5, (2, 4, 0, 1, 3)).reshape(BATCH, SEQ_LEN, D_MODEL)


# speedup vs baseline: 2.1070x; 1.2197x over previous
import functools
import numpy as np
import jax
import jax.numpy as jnp
from jax import lax
from jax.experimental import pallas as pl
from jax.experimental.pallas import tpu as pltpu
from jax.experimental.pallas import tpu_sc as plsc

VOCAB = 100000
D_MODEL = 64
BATCH = 1024
SEQ_LEN = 200

_NC = 2
_NS = 16
_NW = _NC * _NS
_BB = BATCH // 128
_UNITS = SEQ_LEN * _BB
_UPW = _UNITS // _NW
_L = 16
_XROWS = 256


def _positional_encoding(length, depth):
    half = depth / 2
    positions = np.arange(length)[:, np.newaxis]
    depths = np.arange(half)[np.newaxis, :] / half
    angle_rates = 1 / 10000 ** depths
    angle_rads = positions * angle_rates
    pos = np.concatenate([np.sin(angle_rads), np.cos(angle_rads)], axis=-1)
    return pos.astype(np.float32)


def _sc_body(table_hbm, xT_hbm, posP_hbm, out_hbm,
             xall_v, rows_v, outb_v, pos_v, gsems, ssems):
    wid = lax.axis_index("s") * _NC + lax.axis_index("c")
    u0 = wid * _UPW
    l0 = u0 // _BB

    pltpu.sync_copy(xT_hbm.at[pl.ds(l0, 8)], xall_v)
    pltpu.sync_copy(posP_hbm.at[pl.ds(l0, 8)], pos_v)

    def unit_pos(u):
        return u // _BB, lax.rem(u, _BB)

    def gather(u, b):
        l, bt = unit_pos(u)
        return pltpu.make_async_copy(
            table_hbm.at[xall_v.at[l - l0, pl.ds(bt * 128, 128)]],
            rows_v.at[b],
            gsems[b],
        )

    def store(u, b, dt):
        l, bt = unit_pos(u)
        return pltpu.make_async_copy(
            outb_v.at[b, pl.ds(dt * 8, 8)],
            out_hbm.at[l, dt, bt],
            ssems[b],
        )

    def compute(u, b):
        l, bt = unit_pos(u)
        lrow = l - l0
        outb = outb_v.at[b]
        rows = rows_v.at[b]

        iota = lax.iota(jnp.int32, _L)

        def db_body(db, carry):
            d0 = db * _L
            colvec = d0 + iota
            posvec = pos_v[lrow, pl.ds(d0, _L)]
            for rb in range(8):
                r0 = rb * _L
                rot = iota
                for _ in range(4):
                    rvecs, vals = [], []
                    for _ in range(4):
                        rvecs.append(r0 + rot)
                        vals.append(plsc.load_gather(rows, [rvecs[-1], colvec]))
                        rot = lax.bitwise_and(rot + 1, _L - 1)
                    outs = [v * 8.0 + posvec for v in vals]
                    for rv, o in zip(rvecs, outs):
                        plsc.store_scatter(outb, [colvec, rv], o)
            return carry

        lax.fori_loop(0, D_MODEL // _L, db_body, 0)

    gather(u0, 0).start()
    gather(u0 + 1, 1).start()

    def outer(i2, carry):
        for b in range(2):
            ul = i2 * 2 + b
            u = u0 + ul
            gather(u, b).wait()

            @pl.when(ul >= 2)
            def _():
                for dt in range(8):
                    store(u0, b, 0).wait()

            compute(u, b)
            for dt in range(8):
                store(u, b, dt).start()

            @pl.when(ul + 2 < _UPW)
            def _():
                gather(u + 2, b).start()

        return carry

    lax.fori_loop(0, _UPW // 2, outer, 0)
    for b in range(2):
        for dt in range(8):
            store(u0, b, 0).wait()


@jax.jit
def _pos_embed(table, xT, posP):
    mesh = plsc.VectorSubcoreMesh(
        core_axis_name="c", subcore_axis_name="s", num_cores=_NC
    )
    k = pl.kernel(
        _sc_body,
        out_type=jax.ShapeDtypeStruct((SEQ_LEN, 8, 8, 8, 128), jnp.float32),
        mesh=mesh,
        scratch_types=[
            pltpu.VMEM((8, 1024), jnp.int32),
            pltpu.VMEM((2, 128, D_MODEL), jnp.float32),
            pltpu.VMEM((2, D_MODEL, 128), jnp.float32),
            pltpu.VMEM((8, 128), jnp.float32),
            [pltpu.SemaphoreType.DMA] * 2,
            [pltpu.SemaphoreType.DMA] * 2,
        ],
        compiler_params=pltpu.CompilerParams(
            use_tc_tiling_on_sc=False, needs_layout_passes=False
        ),
    )
    return k(table, xT, posP)


def kernel(x, table):
    pos = _positional_encoding(SEQ_LEN, D_MODEL)
    posP = np.zeros((_XROWS, 128), np.float32)
    posP[:SEQ_LEN, :D_MODEL] = pos
    posP = jnp.asarray(posP)
    xT = jnp.pad(jnp.transpose(x.astype(jnp.int32)),
                 ((0, _XROWS - SEQ_LEN), (0, 0)))
    out5 = _pos_embed(table, xT, posP)
    return jnp.transpose(out5, (2, 4, 0, 1, 3)).reshape(BATCH, SEQ_LEN, D_MODEL)


# speedup vs baseline: 2.3182x; 1.1002x over previous
import functools
import numpy as np
import jax
import jax.numpy as jnp
from jax import lax
from jax.experimental import pallas as pl
from jax.experimental.pallas import tpu as pltpu
from jax.experimental.pallas import tpu_sc as plsc

VOCAB = 100000
D_MODEL = 64
BATCH = 1024
SEQ_LEN = 200

_NC = 2
_NS = 16
_NW = _NC * _NS
_BB = BATCH // 128
_UNITS = SEQ_LEN * _BB
_UPW = _UNITS // _NW
_L = 16
_NBUF = 5
_XROWS = 256


def _positional_encoding(length, depth):
    half = depth / 2
    positions = np.arange(length)[:, np.newaxis]
    depths = np.arange(half)[np.newaxis, :] / half
    angle_rates = 1 / 10000 ** depths
    angle_rads = positions * angle_rates
    pos = np.concatenate([np.sin(angle_rads), np.cos(angle_rads)], axis=-1)
    return pos.astype(np.float32)


def _sc_body(table_hbm, xT_hbm, posP_hbm, out_hbm,
             xall_v, rows_v, outb_v, pos_v, gsems, ssems):
    wid = lax.axis_index("s") * _NC + lax.axis_index("c")
    u0 = wid * _UPW
    l0 = u0 // _BB

    pltpu.sync_copy(xT_hbm.at[pl.ds(l0, 8)], xall_v)
    pltpu.sync_copy(posP_hbm.at[pl.ds(l0, 8)], pos_v)

    def unit_pos(u):
        return u // _BB, lax.rem(u, _BB)

    def gather(u, b):
        l, bt = unit_pos(u)
        return pltpu.make_async_copy(
            table_hbm.at[xall_v.at[l - l0, pl.ds(bt * 128, 128)]],
            rows_v.at[b],
            gsems[b],
        )

    def store(u, b, dt):
        l, bt = unit_pos(u)
        return pltpu.make_async_copy(
            outb_v.at[b, pl.ds(dt * 8, 8)],
            out_hbm.at[l, dt, bt],
            ssems[b],
        )

    def compute(u, b):
        l, bt = unit_pos(u)
        lrow = l - l0
        outb = outb_v.at[b]
        rows = rows_v.at[b]

        iota = lax.iota(jnp.int32, _L)

        def db_body(db, carry):
            d0 = db * _L
            colvec = d0 + iota
            posvec = pos_v[lrow, pl.ds(d0, _L)]
            for rb in range(8):
                r0 = rb * _L
                rot = iota
                for _ in range(4):
                    rvecs, vals = [], []
                    for _ in range(4):
                        rvecs.append(r0 + rot)
                        vals.append(plsc.load_gather(rows, [rvecs[-1], colvec]))
                        rot = lax.bitwise_and(rot + 1, _L - 1)
                    outs = [v * 8.0 + posvec for v in vals]
                    for rv, o in zip(rvecs, outs):
                        plsc.store_scatter(outb, [colvec, rv], o)
            return carry

        lax.fori_loop(0, D_MODEL // _L, db_body, 0)

    for b in range(3):
        gather(u0 + b, b).start()

    def outer(i5, carry):
        for b in range(_NBUF):
            ul = i5 * _NBUF + b
            u = u0 + ul
            gather(u, b).wait()

            @pl.when(ul >= _NBUF)
            def _():
                for dt in range(8):
                    store(u0, b, 0).wait()

            compute(u, b)
            for dt in range(8):
                store(u, b, dt).start()

            @pl.when(ul + 3 < _UPW)
            def _():
                gather(u + 3, (b + 3) % _NBUF).start()

        return carry

    lax.fori_loop(0, _UPW // _NBUF, outer, 0)
    for b in range(_NBUF):
        for dt in range(8):
            store(u0, b, 0).wait()


@jax.jit
def _pos_embed(table, xT, posP):
    mesh = plsc.VectorSubcoreMesh(
        core_axis_name="c", subcore_axis_name="s", num_cores=_NC
    )
    k = pl.kernel(
        _sc_body,
        out_type=jax.ShapeDtypeStruct((SEQ_LEN, 8, 8, 8, 128), jnp.float32),
        mesh=mesh,
        scratch_types=[
            pltpu.VMEM((8, 1024), jnp.int32),
            pltpu.VMEM((_NBUF, 128, D_MODEL), jnp.float32),
            pltpu.VMEM((_NBUF, D_MODEL, 128), jnp.float32),
            pltpu.VMEM((8, 128), jnp.float32),
            [pltpu.SemaphoreType.DMA] * _NBUF,
            [pltpu.SemaphoreType.DMA] * _NBUF,
        ],
        compiler_params=pltpu.CompilerParams(
            use_tc_tiling_on_sc=False, needs_layout_passes=False
        ),
    )
    return k(table, xT, posP)


def kernel(x, table):
    pos = _positional_encoding(SEQ_LEN, D_MODEL)
    posP = np.zeros((_XROWS, 128), np.float32)
    posP[:SEQ_LEN, :D_MODEL] = pos
    posP = jnp.asarray(posP)
    xT = jnp.pad(jnp.transpose(x.astype(jnp.int32)),
                 ((0, _XROWS - SEQ_LEN), (0, 0)))
    out5 = _pos_embed(table, xT, posP)
    return jnp.transpose(out5, (2, 4, 0, 1, 3)).reshape(BATCH, SEQ_LEN, D_MODEL)
